# Initial kernel scaffold; baseline (speedup 1.0000x reference)
import jax
import jax.numpy as jnp
from jax.experimental import pallas as pl


def kernel(x, token_table, pos_table):
    raise NotImplementedError("write your pallas kernel here")


# speedup vs baseline: 2.5974x; 2.5974x over previous
import functools

import jax
import jax.numpy as jnp
from jax import lax
from jax.experimental import pallas as pl
from jax.experimental.pallas import tpu as pltpu
from jax.experimental.pallas import tpu_sc as plsc

MAXLEN = 200
EMBED = 64
BATCH = 1024

NUM_CORES = 2
NUM_SUBCORES = 16
NUM_WORKERS = NUM_CORES * NUM_SUBCORES
SEQ_PER_W = BATCH // NUM_WORKERS
LANES = 16


def _make_kernel():
    mesh = plsc.VectorSubcoreMesh(core_axis_name="c", subcore_axis_name="s")

    @functools.partial(
        pl.kernel,
        mesh=mesh,
        out_type=jax.ShapeDtypeStruct((BATCH, MAXLEN, EMBED), jnp.float32),
        scratch_types=[
            pltpu.VMEM((MAXLEN,), jnp.int32),
            pltpu.VMEM((MAXLEN, EMBED), jnp.float32),
            pltpu.VMEM((MAXLEN, EMBED), jnp.float32),
            pltpu.SemaphoreType.DMA,
        ],
        compiler_params=pltpu.CompilerParams(use_tc_tiling_on_sc=False),
    )
    def emb_kernel(x_hbm, tok_hbm, pos_hbm, out_hbm, idx_v, row_v, pos_v, sem):
        wid = lax.axis_index("s") * NUM_CORES + lax.axis_index("c")
        pltpu.sync_copy(pos_hbm, pos_v)

        def seq_body(i, carry):
            seq = wid * SEQ_PER_W + i
            pltpu.sync_copy(x_hbm.at[seq], idx_v)
            pltpu.async_copy(tok_hbm.at[idx_v], row_v, sem).wait()

            def add_body(p, c2):
                for j in range(EMBED // LANES):
                    sl = pl.ds(j * LANES, LANES)
                    row_v[p, sl] = row_v[p, sl] + pos_v[p, sl]
                return c2

            lax.fori_loop(0, MAXLEN, add_body, 0)
            pltpu.sync_copy(row_v, out_hbm.at[seq])
            return carry

        lax.fori_loop(0, SEQ_PER_W, seq_body, 0)

    return emb_kernel


_emb = _make_kernel()


def kernel(x, token_table, pos_table):
    return _emb(x.astype(jnp.int32), token_table, pos_table)


# speedup vs baseline: 3.2333x; 1.2448x over previous
import functools

import jax
import jax.numpy as jnp
from jax import lax
from jax.experimental import pallas as pl
from jax.experimental.pallas import tpu as pltpu
from jax.experimental.pallas import tpu_sc as plsc

MAXLEN = 200
EMBED = 64
BATCH = 1024

NUM_CORES = 2
NUM_SUBCORES = 16
NUM_WORKERS = NUM_CORES * NUM_SUBCORES
SEQ_PER_W = BATCH // NUM_WORKERS
LANES = 16

GRP = 2
NGRP = SEQ_PER_W // GRP
NBUF = 4
LEAD = 2
LAG = 2


def _make_kernel():
    mesh = plsc.VectorSubcoreMesh(core_axis_name="c", subcore_axis_name="s")

    @functools.partial(
        pl.kernel,
        mesh=mesh,
        out_type=jax.ShapeDtypeStruct((BATCH, MAXLEN, EMBED), jnp.float32),
        scratch_types=[
            [pltpu.VMEM((MAXLEN,), jnp.int32)] * SEQ_PER_W,
            pltpu.VMEM((MAXLEN, EMBED), jnp.float32),
            [pltpu.VMEM((GRP, MAXLEN, EMBED), jnp.float32)] * NBUF,
            [pltpu.SemaphoreType.DMA] * NBUF,
            [pltpu.SemaphoreType.DMA] * NBUF,
        ],
        compiler_params=pltpu.CompilerParams(use_tc_tiling_on_sc=False),
    )
    def emb_kernel(x_hbm, tok_hbm, pos_hbm, out_hbm, idx_v, pos_v, bufs,
                   gsems, ssems):
        wid = lax.axis_index("s") * NUM_CORES + lax.axis_index("c")
        seq0 = wid * SEQ_PER_W
        for s in range(SEQ_PER_W):
            pltpu.async_copy(x_hbm.at[seq0 + s], idx_v[s], gsems[0])
        for s in range(SEQ_PER_W):
            pltpu.make_async_copy(x_hbm.at[seq0 + s], idx_v[s], gsems[0]).wait()
        pltpu.sync_copy(pos_hbm, pos_v)

        def fire_gathers(t):
            b = t % NBUF
            for k in range(GRP):
                pltpu.async_copy(
                    tok_hbm.at[idx_v[t * GRP + k]], bufs[b].at[k], gsems[b])

        def add_group(t):
            b = t % NBUF
            buf = bufs[b]

            def body(p, c):
                pos_regs = [pos_v[p, pl.ds(j * LANES, LANES)]
                            for j in range(EMBED // LANES)]
                for k in range(GRP):
                    for j in range(EMBED // LANES):
                        sl = pl.ds(j * LANES, LANES)
                        buf[k, p, sl] = buf[k, p, sl] + pos_regs[j]
                return c

            lax.fori_loop(0, MAXLEN, body, 0)

        for t in range(LEAD):
            fire_gathers(t)
        for t in range(NGRP):
            b = t % NBUF
            for k in range(GRP):
                pltpu.make_async_copy(
                    tok_hbm.at[idx_v[t * GRP + k]], bufs[b].at[k],
                    gsems[b]).wait()
            add_group(t)
            pltpu.async_copy(
                bufs[b], out_hbm.at[pl.ds(seq0 + t * GRP, GRP)], ssems[b])
            if t >= LAG:
                ob = (t - LAG) % NBUF
                pltpu.make_async_copy(
                    bufs[ob],
                    out_hbm.at[pl.ds(seq0 + (t - LAG) * GRP, GRP)],
                    ssems[ob]).wait()
            if t + LEAD < NGRP:
                fire_gathers(t + LEAD)
        for t in range(NGRP - LAG, NGRP):
            b = t % NBUF
            pltpu.make_async_copy(
                bufs[b], out_hbm.at[pl.ds(seq0 + t * GRP, GRP)],
                ssems[b]).wait()

    return emb_kernel


_emb = _make_kernel()


def kernel(x, token_table, pos_table):
    return _emb(x.astype(jnp.int32), token_table, pos_table)


# speedup vs baseline: 4.7164x; 1.4587x over previous
import functools

import jax
import jax.numpy as jnp
from jax import lax
from jax.experimental import pallas as pl
from jax.experimental.pallas import tpu as pltpu
from jax.experimental.pallas import tpu_sc as plsc

MAXLEN = 200
EMBED = 64
BATCH = 1024
OUTMIN = 128

NUM_CORES = 2
NUM_SUBCORES = 16
NUM_WORKERS = NUM_CORES * NUM_SUBCORES
SEQ_PER_W = BATCH // NUM_WORKERS
LANES = 16

GRP = 2
NGRP = SEQ_PER_W // GRP
NBUF = 4
LEAD = 2
LAG = 2


def _make_kernel():
    mesh = plsc.VectorSubcoreMesh(core_axis_name="c", subcore_axis_name="s")

    @functools.partial(
        pl.kernel,
        mesh=mesh,
        out_type=jax.ShapeDtypeStruct((BATCH, MAXLEN, OUTMIN), jnp.float32),
        scratch_types=[
            [pltpu.VMEM((MAXLEN,), jnp.int32)] * SEQ_PER_W,
            pltpu.VMEM((MAXLEN * EMBED,), jnp.float32),
            [pltpu.VMEM((GRP, MAXLEN, EMBED), jnp.float32)] * NBUF,
            [pltpu.SemaphoreType.DMA] * NBUF,
            [pltpu.SemaphoreType.DMA] * NBUF,
        ],
        compiler_params=pltpu.CompilerParams(use_tc_tiling_on_sc=False),
    )
    def emb_kernel(x_hbm, tok_hbm, pos_hbm, out_hbm, idx_v, pos_v, bufs,
                   gsems, ssems):
        wid = lax.axis_index("s") * NUM_CORES + lax.axis_index("c")
        seq0 = wid * SEQ_PER_W
        for s in range(SEQ_PER_W):
            pltpu.async_copy(
                x_hbm.at[pl.ds((seq0 + s) * MAXLEN, MAXLEN)], idx_v[s],
                gsems[0])
        for s in range(SEQ_PER_W):
            pltpu.make_async_copy(
                x_hbm.at[pl.ds((seq0 + s) * MAXLEN, MAXLEN)], idx_v[s],
                gsems[0]).wait()
        pltpu.sync_copy(pos_hbm, pos_v)

        def fire_gathers(t):
            b = t % NBUF
            for k in range(GRP):
                pltpu.async_copy(
                    tok_hbm.at[idx_v[t * GRP + k]], bufs[b].at[k], gsems[b])

        def add_group(t):
            b = t % NBUF
            buf = bufs[b]

            def body(p, c):
                pos_regs = [pos_v[pl.ds(p * EMBED + j * LANES, LANES)]
                            for j in range(EMBED // LANES)]
                for k in range(GRP):
                    for j in range(EMBED // LANES):
                        sl = pl.ds(j * LANES, LANES)
                        buf[k, p, sl] = buf[k, p, sl] + pos_regs[j]
                return c

            lax.fori_loop(0, MAXLEN, body, 0)

        def out_slice(t):
            return out_hbm.at[pl.ds(seq0 + t * GRP, GRP), :, pl.ds(0, EMBED)]

        for t in range(LEAD):
            fire_gathers(t)
        for t in range(NGRP):
            b = t % NBUF
            for k in range(GRP):
                pltpu.make_async_copy(
                    tok_hbm.at[idx_v[t * GRP + k]], bufs[b].at[k],
                    gsems[b]).wait()
            add_group(t)
            pltpu.async_copy(bufs[b], out_slice(t), ssems[b])
            if t >= LAG:
                ob = (t - LAG) % NBUF
                pltpu.make_async_copy(
                    bufs[ob], out_slice(t - LAG), ssems[ob]).wait()
            if t + LEAD < NGRP:
                fire_gathers(t + LEAD)
        for t in range(NGRP - LAG, NGRP):
            b = t % NBUF
            pltpu.make_async_copy(bufs[b], out_slice(t), ssems[b]).wait()

    return emb_kernel


_emb = _make_kernel()


def kernel(x, token_table, pos_table):
    out = _emb(x.reshape(-1).astype(jnp.int32), token_table,
               pos_table.reshape(-1))
    return out[:, :, :EMBED]
